# baseline (speedup 1.0000x reference)
import functools

import jax
import jax.numpy as jnp
from jax import lax
from jax.experimental import pallas as pl
from jax.experimental.pallas import tpu as pltpu
from jax.experimental.pallas import tpu_sc as plsc

VOCAB = 100000
HIDDEN = 128
MAX_POS = 2048
B, T = 4, 2048
N_ROWS = B * T

_CHUNK = 64


def _make_sc_kernel():
    info = plsc.get_sparse_core_info()
    nc, ns = info.num_cores, info.num_subcores
    nw = nc * ns
    rows_w = N_ROWS // nw
    n_chunks = rows_w // _CHUNK

    mesh = plsc.VectorSubcoreMesh(core_axis_name="c", subcore_axis_name="s")

    @functools.partial(
        pl.kernel,
        mesh=mesh,
        out_type=jax.ShapeDtypeStruct((B, T, HIDDEN), jnp.float32),
        scratch_types=[
            pltpu.VMEM((n_chunks, _CHUNK), jnp.int32),
            pltpu.VMEM((rows_w, HIDDEN), jnp.float32),
        ]
        + [pltpu.SemaphoreType.DMA] * (2 + 2 * n_chunks),
    )
    def sc_kernel(ids_hbm, tok_hbm, pos_hbm, out_hbm, idx_v, tok_v, *sems):
        sem_i, sem_p = sems[0], sems[1]
        sem_g = sems[2 : 2 + n_chunks]
        sem_o = sems[2 + n_chunks :]

        wid = lax.axis_index("s") * nc + lax.axis_index("c")
        base = wid * rows_w
        b = base // T
        col = lax.rem(base, T)

        idx_cp = [
            pltpu.async_copy(
                ids_hbm.at[b, pl.ds(col + c * _CHUNK, _CHUNK)],
                idx_v.at[c],
                sem_i,
            )
            for c in range(n_chunks)
        ]
        pos_cp = pltpu.async_copy(pos_hbm.at[pl.ds(col, rows_w)], tok_v, sem_p)
        for cp in idx_cp:
            cp.wait()
        pos_cp.wait()

        g_cp = [
            pltpu.async_copy(
                tok_hbm.at[idx_v.at[c]],
                tok_v.at[pl.ds(c * _CHUNK, _CHUNK)],
                sem_g[c],
                add=True,
            )
            for c in range(n_chunks)
        ]
        out_cp = []
        for c in range(n_chunks):
            g_cp[c].wait()
            out_cp.append(
                pltpu.async_copy(
                    tok_v.at[pl.ds(c * _CHUNK, _CHUNK)],
                    out_hbm.at[b, pl.ds(col + c * _CHUNK, _CHUNK)],
                    sem_o[c],
                )
            )
        for cp in out_cp:
            cp.wait()

    return sc_kernel


def kernel(input_ids, token_table, pos_table):
    return _make_sc_kernel()(
        input_ids.astype(jnp.int32), token_table, pos_table
    )


# speedup vs baseline: 1.0266x; 1.0266x over previous
import functools

import jax
import jax.numpy as jnp
from jax import lax
from jax.experimental import pallas as pl
from jax.experimental.pallas import tpu as pltpu
from jax.experimental.pallas import tpu_sc as plsc

VOCAB = 100000
HIDDEN = 128
MAX_POS = 2048
B, T = 4, 2048
N_ROWS = B * T

_CHUNK = 128


def _make_sc_kernel():
    info = plsc.get_sparse_core_info()
    nc, ns = info.num_cores, info.num_subcores
    nw = nc * ns
    rows_w = N_ROWS // nw
    n_chunks = rows_w // _CHUNK

    mesh = plsc.VectorSubcoreMesh(core_axis_name="c", subcore_axis_name="s")

    @functools.partial(
        pl.kernel,
        mesh=mesh,
        out_type=jax.ShapeDtypeStruct((B, T, HIDDEN), jnp.float32),
        scratch_types=[
            pltpu.VMEM((n_chunks, _CHUNK), jnp.int32),
            pltpu.VMEM((rows_w, HIDDEN), jnp.float32),
        ]
        + [pltpu.SemaphoreType.DMA] * (4 * n_chunks),
    )
    def sc_kernel(ids_hbm, tok_hbm, pos_hbm, out_hbm, idx_v, tok_v, *sems):
        sem_i = sems[:n_chunks]
        sem_p = sems[n_chunks : 2 * n_chunks]
        sem_g = sems[2 * n_chunks : 3 * n_chunks]
        sem_o = sems[3 * n_chunks :]

        wid = lax.axis_index("s") * nc + lax.axis_index("c")
        base = wid * rows_w
        b = base // T
        col = lax.rem(base, T)

        def chunk(ref, c):
            return ref.at[pl.ds(c * _CHUNK, _CHUNK)]

        idx_cp = [
            pltpu.async_copy(
                ids_hbm.at[b, pl.ds(col + c * _CHUNK, _CHUNK)],
                idx_v.at[c],
                sem_i[c],
            )
            for c in range(n_chunks)
        ]
        pos_cp = [
            pltpu.async_copy(
                pos_hbm.at[pl.ds(col + c * _CHUNK, _CHUNK)],
                chunk(tok_v, c),
                sem_p[c],
            )
            for c in range(n_chunks)
        ]

        g_cp = []
        for c in range(n_chunks):
            idx_cp[c].wait()
            pos_cp[c].wait()
            g_cp.append(
                pltpu.async_copy(
                    tok_hbm.at[idx_v.at[c]],
                    chunk(tok_v, c),
                    sem_g[c],
                    add=True,
                )
            )
        out_cp = []
        for c in range(n_chunks):
            g_cp[c].wait()
            out_cp.append(
                pltpu.async_copy(
                    chunk(tok_v, c),
                    out_hbm.at[b, pl.ds(col + c * _CHUNK, _CHUNK)],
                    sem_o[c],
                )
            )
        for cp in out_cp:
            cp.wait()

    return sc_kernel


def kernel(input_ids, token_table, pos_table):
    return _make_sc_kernel()(
        input_ids.astype(jnp.int32), token_table, pos_table
    )
